# baseline (speedup 1.0000x reference)
import functools

import jax
import jax.numpy as jnp
from jax import lax
from jax.experimental import pallas as pl
from jax.experimental.pallas import tpu as pltpu
from jax.experimental.pallas import tpu_sc as plsc

VOCAB = 1000
EMB = 64
B = 1024
VPAD = 1024

NC = 2
NS = 16
NW = NC * NS
RPW = B // NW
G = 16
GROUPS = RPW // G
UNROLL = 8


def _tc_p_body(emb_v_ref, emb_u_ref, p_ref):
    p_ref[...] = lax.dot_general(
        emb_v_ref[...], emb_u_ref[...], (((1,), (1,)), ((), ())),
        preferred_element_type=jnp.float32, precision=lax.Precision.HIGHEST)


def _sc_sumexp_body(p_hbm, cen_hbm, tgt_hbm, av_hbm, se_hbm, sco_hbm,
                    cen_v, tgt_v, rows_v, av_v, se_v, sco_v, sem_r, sem_a):
    cid = lax.axis_index("c")
    sid = lax.axis_index("s")
    wid = sid * NC + cid
    base = wid * RPW
    lanes = lax.broadcasted_iota(jnp.int32, (G,), 0)

    pltpu.sync_copy(cen_hbm.at[pl.ds(base, RPW)], cen_v)
    pltpu.sync_copy(tgt_hbm.at[pl.ds(base, RPW)], tgt_v)
    cp_rows = pltpu.async_copy(p_hbm.at[cen_v], rows_v, sem_r)
    cp_av = pltpu.async_copy(av_hbm.at[pl.ds(base, RPW)], av_v, sem_a)
    cp_rows.wait()
    cp_av.wait()

    def step(j0, accs):
        new = list(accs)
        for u in range(UNROLL):
            col = jnp.full((G,), j0 * UNROLL + u, dtype=jnp.int32)
            for g in range(GROUPS):
                gl = lanes + g * G
                iv = plsc.load_gather(av_v, [gl, col])
                vals = plsc.load_gather(rows_v, [gl, iv])
                new[g] = new[g] + jnp.exp(vals)
        return tuple(new)

    zero = jnp.zeros((G,), jnp.float32)
    accs = lax.fori_loop(0, VOCAB // UNROLL, step, (zero,) * GROUPS)

    for g in range(GROUPS):
        se_v[pl.ds(g * G, G)] = accs[g]
        gl = lanes + g * G
        tv = tgt_v[pl.ds(g * G, G)]
        sco_v[pl.ds(g * G, G)] = plsc.load_gather(rows_v, [gl, tv])
    pltpu.sync_copy(se_v, se_hbm.at[pl.ds(base, RPW)])
    pltpu.sync_copy(sco_v, sco_hbm.at[pl.ds(base, RPW)])


def _tc_final_body(sumexp_ref, scores_ref, o_ref):
    nll = jnp.mean(jnp.log(sumexp_ref[...])) - jnp.mean(scores_ref[...])
    o_ref[...] = nll.reshape(1, 1)


@jax.jit
def kernel(center_words, target_words, all_vocabs, emb_v, emb_u):
    emb_u_pad = jnp.pad(emb_u, ((0, VPAD - VOCAB), (0, 0)))

    p = pl.pallas_call(
        _tc_p_body,
        out_shape=jax.ShapeDtypeStruct((VOCAB, VPAD), jnp.float32),
    )(emb_v, emb_u_pad)

    sumexp, scores = pl.kernel(
        _sc_sumexp_body,
        mesh=plsc.VectorSubcoreMesh(core_axis_name="c", subcore_axis_name="s"),
        out_type=[
            jax.ShapeDtypeStruct((B,), jnp.float32),
            jax.ShapeDtypeStruct((B,), jnp.float32),
        ],
        scratch_types=[
            pltpu.VMEM((RPW,), jnp.int32),
            pltpu.VMEM((RPW,), jnp.int32),
            pltpu.VMEM((RPW, VPAD), jnp.float32),
            pltpu.VMEM((RPW, VOCAB), jnp.int32),
            pltpu.VMEM((RPW,), jnp.float32),
            pltpu.VMEM((RPW,), jnp.float32),
            pltpu.SemaphoreType.DMA,
            pltpu.SemaphoreType.DMA,
        ],
        compiler_params=pltpu.CompilerParams(
            use_tc_tiling_on_sc=True, needs_layout_passes=False),
    )(p, center_words.reshape(B), target_words.reshape(B), all_vocabs)

    nll = pl.pallas_call(
        _tc_final_body,
        out_shape=jax.ShapeDtypeStruct((1, 1), jnp.float32),
    )(sumexp.reshape(8, 128), scores.reshape(8, 128))
    return nll[0, 0]


# speedup vs baseline: 1.3088x; 1.3088x over previous
import functools

import jax
import jax.numpy as jnp
from jax import lax
from jax.experimental import pallas as pl
from jax.experimental.pallas import tpu as pltpu
from jax.experimental.pallas import tpu_sc as plsc

VOCAB = 1000
EMB = 64
B = 1024
NCHUNK = 8
CSTRIDE = 1024
PLROWS = NCHUNK * CSTRIDE

NC = 2
NS = 16
NW = NC * NS
RPW = B // NW
G = 16
GROUPS = RPW // G
UNROLL = 8


def _tc_p_body(emb_v_ref, emb_u_ref, pl_ref):
    p = lax.dot_general(
        emb_v_ref[...], emb_u_ref[...], (((1,), (1,)), ((), ())),
        preferred_element_type=jnp.float32, precision=lax.Precision.HIGHEST)
    for ct in range(NCHUNK):
        w = min(128, VOCAB - ct * 128)
        pl_ref[pl.ds(ct * CSTRIDE, VOCAB), pl.ds(0, w)] = (
            p[:, ct * 128:ct * 128 + w])


def _sc_sumexp_body(pl_hbm, cen_hbm, tgt_hbm, av_hbm, se_hbm, sco_hbm,
                    cen_v, tgt_v, idx_a, idx_b, rows_v, av_v, se_v, sco_v,
                    sem_r, sem_s, sem_a):
    cid = lax.axis_index("c")
    sid = lax.axis_index("s")
    wid = sid * NC + cid
    base = wid * RPW
    lanes = lax.broadcasted_iota(jnp.int32, (G,), 0)

    pltpu.sync_copy(cen_hbm.at[pl.ds(base, RPW)], cen_v)
    pltpu.sync_copy(tgt_hbm.at[pl.ds(base, RPW)], tgt_v)
    cp_av = pltpu.async_copy(
        av_hbm.at[pl.ds(base * VOCAB, RPW * VOCAB)], av_v, sem_a)

    ct_off = (lanes & 7) * CSTRIDE
    b_sel = lanes >> 3
    for i in range(8):
        idx_a[pl.ds(i * G, G)] = (
            plsc.load_gather(cen_v, [b_sel + 2 * i]) + ct_off)
        idx_b[pl.ds(i * G, G)] = (
            plsc.load_gather(cen_v, [b_sel + 2 * i + G]) + ct_off)
    cp_r = pltpu.async_copy(pl_hbm.at[idx_a], rows_v.at[pl.ds(0, 128)], sem_r)
    cp_s = pltpu.async_copy(pl_hbm.at[idx_b], rows_v.at[pl.ds(128, 128)],
                            sem_s)
    cp_r.wait()
    cp_s.wait()
    cp_av.wait()

    b8 = [(lanes + g * G) * 8 for g in range(GROUPS)]
    b1000 = [(lanes + g * G) * VOCAB for g in range(GROUPS)]

    def step(j0, accs):
        new = list(accs)
        for u in range(UNROLL):
            j = j0 * UNROLL + u
            for g in range(GROUPS):
                iv = plsc.load_gather(av_v, [b1000[g] + j])
                vals = plsc.load_gather(
                    rows_v, [b8[g] + (iv >> 7), iv & 127])
                new[g] = new[g] + jnp.exp(vals)
        return tuple(new)

    zero = jnp.zeros((G,), jnp.float32)
    accs = lax.fori_loop(0, VOCAB // UNROLL, step, (zero,) * GROUPS)

    for g in range(GROUPS):
        se_v[pl.ds(g * G, G)] = accs[g]
        tv = tgt_v[pl.ds(g * G, G)]
        sco_v[pl.ds(g * G, G)] = plsc.load_gather(
            rows_v, [b8[g] + (tv >> 7), tv & 127])
    pltpu.sync_copy(se_v, se_hbm.at[pl.ds(base, RPW)])
    pltpu.sync_copy(sco_v, sco_hbm.at[pl.ds(base, RPW)])


def _tc_final_body(sumexp_ref, scores_ref, o_ref):
    nll = jnp.mean(jnp.log(sumexp_ref[...])) - jnp.mean(scores_ref[...])
    o_ref[...] = nll.reshape(1, 1)


@jax.jit
def kernel(center_words, target_words, all_vocabs, emb_v, emb_u):
    pl_mat = pl.pallas_call(
        _tc_p_body,
        out_shape=jax.ShapeDtypeStruct((PLROWS, 128), jnp.float32),
    )(emb_v, emb_u)

    sumexp, scores = pl.kernel(
        _sc_sumexp_body,
        mesh=plsc.VectorSubcoreMesh(core_axis_name="c", subcore_axis_name="s"),
        out_type=[
            jax.ShapeDtypeStruct((B,), jnp.float32),
            jax.ShapeDtypeStruct((B,), jnp.float32),
        ],
        scratch_types=[
            pltpu.VMEM((RPW,), jnp.int32),
            pltpu.VMEM((RPW,), jnp.int32),
            pltpu.VMEM((128,), jnp.int32),
            pltpu.VMEM((128,), jnp.int32),
            pltpu.VMEM((2 * 128, 128), jnp.float32),
            pltpu.VMEM((RPW * VOCAB,), jnp.int32),
            pltpu.VMEM((RPW,), jnp.float32),
            pltpu.VMEM((RPW,), jnp.float32),
            pltpu.SemaphoreType.DMA,
            pltpu.SemaphoreType.DMA,
            pltpu.SemaphoreType.DMA,
        ],
        compiler_params=pltpu.CompilerParams(
            use_tc_tiling_on_sc=False, needs_layout_passes=False),
    )(pl_mat, center_words.reshape(B), target_words.reshape(B),
      all_vocabs.reshape(B * VOCAB))

    nll = pl.pallas_call(
        _tc_final_body,
        out_shape=jax.ShapeDtypeStruct((1, 1), jnp.float32),
    )(sumexp.reshape(8, 128), scores.reshape(8, 128))
    return nll[0, 0]
